# baseline (speedup 1.0000x reference)
import jax
import jax.numpy as jnp
from jax.experimental import pallas as pl

_BK = 256


def _router_kernel(x_ref, w_ref, o_ref):
    k = pl.program_id(0)
    nk = pl.num_programs(0)
    part = jax.lax.dot_general(
        x_ref[...].astype(jnp.bfloat16), w_ref[...].astype(jnp.bfloat16),
        dimension_numbers=(((1,), (1,)), ((), ())),
        preferred_element_type=jnp.float32,
    )

    @pl.when(k == 0)
    def _():
        o_ref[...] = part

    @pl.when(jnp.logical_and(k > 0, k < nk - 1))
    def _():
        o_ref[...] += part

    @pl.when(jnp.logical_and(k == nk - 1, nk > 1))
    def _():
        o_ref[...] = jnp.maximum(o_ref[...] + part, 0.0)


def kernel(x, W):
    M, K = x.shape
    E = W.shape[0]
    return pl.pallas_call(
        _router_kernel,
        grid=(K // _BK,),
        in_specs=[
            pl.BlockSpec((M, _BK), lambda k: (0, k)),
            pl.BlockSpec((E, _BK), lambda k: (0, k)),
        ],
        out_specs=pl.BlockSpec((M, E), lambda k: (0, 0)),
        out_shape=jax.ShapeDtypeStruct((M, E), x.dtype),
    )(x, W)


# speedup vs baseline: 1.0583x; 1.0583x over previous
import jax
import jax.numpy as jnp
from jax.experimental import pallas as pl
from jax.experimental.pallas import tpu as pltpu


def _router_kernel(x_ref, w_ref, o_ref):
    logits = jax.lax.dot_general(
        x_ref[...].astype(jnp.bfloat16), w_ref[...].astype(jnp.bfloat16),
        dimension_numbers=(((1,), (1,)), ((), ())),
        preferred_element_type=jnp.float32,
    )
    o_ref[...] = jnp.maximum(logits, 0.0)


def kernel(x, W):
    M, K = x.shape
    E = W.shape[0]
    BM = 1024
    return pl.pallas_call(
        _router_kernel,
        grid=(M // BM,),
        in_specs=[
            pl.BlockSpec((BM, K), lambda i: (i, 0)),
            pl.BlockSpec((E, K), lambda i: (0, 0)),
        ],
        out_specs=pl.BlockSpec((BM, E), lambda i: (i, 0)),
        out_shape=jax.ShapeDtypeStruct((M, E), x.dtype),
        compiler_params=pltpu.CompilerParams(
            dimension_semantics=("parallel",),
        ),
    )(x, W)


# speedup vs baseline: 1.0609x; 1.0025x over previous
import jax
import jax.numpy as jnp
from jax.experimental import pallas as pl

_BM = 1024


def _router_kernel(x_ref, w_ref, o_ref):
    logits = jax.lax.dot_general(
        x_ref[0].astype(jnp.bfloat16), w_ref[...].astype(jnp.bfloat16),
        dimension_numbers=(((1,), (1,)), ((), ())),
        preferred_element_type=jnp.float32,
    )
    o_ref[0] = jnp.maximum(logits, 0.0)


def kernel(x, W):
    M, K = x.shape
    E = W.shape[0]
    nblk = M // _BM
    x3 = x.reshape(nblk, _BM, K)
    out = pl.pallas_call(
        _router_kernel,
        grid=(nblk,),
        in_specs=[
            pl.BlockSpec((1, _BM, K), lambda i: (i, 0, 0)),
            pl.BlockSpec((E, K), lambda i: (0, 0)),
        ],
        out_specs=pl.BlockSpec((1, _BM, E), lambda i: (i, 0, 0)),
        out_shape=jax.ShapeDtypeStruct((nblk, _BM, E), x.dtype),
    )(x3, W)
    return out.reshape(M, E)
